# baseline (speedup 1.0000x reference)
import functools
import math

import jax
import jax.numpy as jnp
from jax import lax
from jax.experimental import pallas as pl
from jax.experimental.pallas import tpu as pltpu
from jax.experimental.pallas import tpu_sc as plsc

D_MODEL = 64
SCALE = math.sqrt(D_MODEL)

NUM_CORES = 2
NUM_SUBCORES = 16
NUM_WORKERS = NUM_CORES * NUM_SUBCORES

CHUNK = 128
NBUF = 4
TBUF = 4


def _make_lookup(batch, seqlen):
    bw = batch // NUM_WORKERS
    assert bw == CHUNK
    n_groups = seqlen // NBUF
    assert n_groups * NBUF == seqlen
    jblk = batch // 128

    mesh = plsc.VectorSubcoreMesh(
        core_axis_name="c", subcore_axis_name="s",
        num_cores=NUM_CORES, num_subcores=NUM_SUBCORES)

    @functools.partial(
        pl.kernel,
        out_type=jax.ShapeDtypeStruct(
            (seqlen, D_MODEL // 8, jblk, 8, 128), jnp.float32),
        mesh=mesh,
        compiler_params=pltpu.CompilerParams(
            use_tc_tiling_on_sc=False, needs_layout_passes=False),
        scratch_types=[
            pltpu.VMEM((seqlen, CHUNK), jnp.int32),
            pltpu.VMEM((NBUF, CHUNK, D_MODEL), jnp.float32),
            pltpu.VMEM((TBUF, D_MODEL // 8, 8, 128), jnp.float32),
            pltpu.SemaphoreType.DMA,
            pltpu.SemaphoreType.DMA,
        ],
    )
    def lookup_kernel(tokt_hbm, table_hbm, out_hbm, tok_v, rows_v, t_v,
                      gsem, osem):
        wid = lax.axis_index("s") * NUM_CORES + lax.axis_index("c")
        pltpu.sync_copy(tokt_hbm.at[:, pl.ds(wid * bw, bw)], tok_v)
        lane = lax.iota(jnp.int32, 16)
        row_idx = [lane + 16 * g for g in range(8)]

        def start_gather(s, slot):
            pltpu.make_async_copy(
                table_hbm.at[tok_v.at[s]], rows_v.at[slot], gsem).start()

        for u in range(NBUF):
            start_gather(u, u)

        def group(gr, _):
            s0 = gr * NBUF
            for u in range(NBUF):
                s = s0 + u
                pltpu.make_async_copy(
                    table_hbm.at[tok_v.at[s]], rows_v.at[u], gsem).wait()
                @pl.when(gr > 0)
                def _wait_prev_out():
                    pltpu.make_async_copy(
                        t_v.at[u], out_hbm.at[s0 - NBUF + u, :, wid],
                        osem).wait()
                for g in range(8):
                    tok16 = tok_v[s, pl.ds(16 * g, 16)]
                    scale = jnp.where(tok16 == 0, 0.0, jnp.float32(SCALE))
                    for d in range(D_MODEL):
                        val = plsc.load_gather(
                            rows_v.at[u],
                            [row_idx[g], jnp.full((16,), d, jnp.int32)])
                        t_v[u, d // 8, d % 8, pl.ds(16 * g, 16)] = val * scale
                pltpu.make_async_copy(
                    t_v.at[u], out_hbm.at[s, :, wid], osem).start()
                @pl.when(s + NBUF < seqlen)
                def _refill():
                    start_gather(s + NBUF, u)
            return 0

        lax.fori_loop(0, n_groups, group, 0)
        for u in range(NBUF):
            s = seqlen - NBUF + u
            pltpu.make_async_copy(
                t_v.at[u], out_hbm.at[s, :, wid], osem).wait()

    return lookup_kernel


def kernel(tokens, table):
    batch, seqlen = tokens.shape
    tokens_t = tokens.T.astype(jnp.int32)
    out5 = _make_lookup(batch, seqlen)(tokens_t, table)
    out_t = out5.transpose(0, 1, 3, 2, 4).reshape(seqlen, D_MODEL, batch)
    return out_t.transpose(2, 0, 1)


# speedup vs baseline: 1.4803x; 1.4803x over previous
import functools
import math

import jax
import jax.numpy as jnp
from jax import lax
from jax.experimental import pallas as pl
from jax.experimental.pallas import tpu as pltpu
from jax.experimental.pallas import tpu_sc as plsc

D_MODEL = 64
SCALE = math.sqrt(D_MODEL)

NUM_CORES = 2
NUM_SUBCORES = 16
NUM_WORKERS = NUM_CORES * NUM_SUBCORES

CHUNK = 128
NBUF = 4
TBUF = 4


def _make_lookup(batch, seqlen):
    bw = batch // NUM_WORKERS
    assert bw == CHUNK
    n_groups = seqlen // NBUF
    assert n_groups * NBUF == seqlen
    jblk = batch // 128

    mesh = plsc.VectorSubcoreMesh(
        core_axis_name="c", subcore_axis_name="s",
        num_cores=NUM_CORES, num_subcores=NUM_SUBCORES)

    @functools.partial(
        pl.kernel,
        out_type=jax.ShapeDtypeStruct(
            (seqlen, D_MODEL // 8, jblk, 8, 128), jnp.float32),
        mesh=mesh,
        compiler_params=pltpu.CompilerParams(
            use_tc_tiling_on_sc=False, needs_layout_passes=False),
        scratch_types=[
            pltpu.VMEM((seqlen, CHUNK), jnp.int32),
            pltpu.VMEM((NBUF, CHUNK, D_MODEL + 1), jnp.float32),
            pltpu.VMEM((TBUF, D_MODEL // 8, 8, 128), jnp.float32),
            pltpu.SemaphoreType.DMA,
            pltpu.SemaphoreType.DMA,
        ],
    )
    def lookup_kernel(tokt_hbm, table_hbm, out_hbm, tok_v, rows_v, t_v,
                      gsem, osem):
        wid = lax.axis_index("s") * NUM_CORES + lax.axis_index("c")
        pltpu.sync_copy(tokt_hbm.at[:, pl.ds(wid * bw, bw)], tok_v)
        lane = lax.iota(jnp.int32, 16)
        row_idx = [lane + 16 * g for g in range(8)]

        def start_gather(s, slot):
            pltpu.make_async_copy(
                table_hbm.at[tok_v.at[s]], rows_v.at[slot], gsem).start()

        for u in range(NBUF):
            start_gather(u, u)

        def group(gr, _):
            s0 = gr * NBUF
            for u in range(NBUF):
                s = s0 + u
                pltpu.make_async_copy(
                    table_hbm.at[tok_v.at[s]], rows_v.at[u], gsem).wait()
                @pl.when(gr > 0)
                def _wait_prev_out():
                    pltpu.make_async_copy(
                        t_v.at[u], out_hbm.at[s0 - NBUF + u, :, wid],
                        osem).wait()
                for g in range(8):
                    tok16 = tok_v[s, pl.ds(16 * g, 16)]
                    scale = jnp.where(tok16 == 0, 0.0, jnp.float32(SCALE))
                    for d in range(D_MODEL):
                        val = plsc.load_gather(
                            rows_v.at[u],
                            [row_idx[g], jnp.full((16,), d, jnp.int32)])
                        t_v[u, d // 8, d % 8, pl.ds(16 * g, 16)] = val * scale
                pltpu.make_async_copy(
                    t_v.at[u], out_hbm.at[s, :, wid], osem).start()
                @pl.when(s + NBUF < seqlen)
                def _refill():
                    start_gather(s + NBUF, u)
            return 0

        lax.fori_loop(0, n_groups, group, 0)
        for u in range(NBUF):
            s = seqlen - NBUF + u
            pltpu.make_async_copy(
                t_v.at[u], out_hbm.at[s, :, wid], osem).wait()

    return lookup_kernel


def kernel(tokens, table):
    batch, seqlen = tokens.shape
    tokens_t = tokens.T.astype(jnp.int32)
    tab65 = jnp.pad(table, ((0, 0), (0, 1)))
    out5 = _make_lookup(batch, seqlen)(tokens_t, tab65)
    out_t = out5.transpose(0, 1, 3, 2, 4).reshape(seqlen, D_MODEL, batch)
    return out_t.transpose(2, 0, 1)


# speedup vs baseline: 3.6640x; 2.4752x over previous
import functools
import math

import jax
import jax.numpy as jnp
from jax import lax
from jax.experimental import pallas as pl
from jax.experimental.pallas import tpu as pltpu
from jax.experimental.pallas import tpu_sc as plsc

D_MODEL = 64
SCALE = math.sqrt(D_MODEL)

NUM_CORES = 2
NUM_SUBCORES = 16
NUM_WORKERS = NUM_CORES * NUM_SUBCORES

CHUNK = 128
NBUF = 8

B_BLK = 512
Q_BLK = 4


def _make_gather(batch, seqlen):
    num_tokens = batch * seqlen
    per_worker = num_tokens // NUM_WORKERS
    bw = batch // NUM_WORKERS
    assert bw == CHUNK and seqlen % 2 == 0
    n_chunks = per_worker // CHUNK
    assert NBUF % 2 == 0 and n_chunks % NBUF == 0
    n_groups = n_chunks // NBUF

    mesh = plsc.VectorSubcoreMesh(
        core_axis_name="c", subcore_axis_name="s",
        num_cores=NUM_CORES, num_subcores=NUM_SUBCORES)

    @functools.partial(
        pl.kernel,
        out_type=jax.ShapeDtypeStruct((num_tokens, D_MODEL), jnp.float32),
        mesh=mesh,
        compiler_params=pltpu.CompilerParams(
            use_tc_tiling_on_sc=False, needs_layout_passes=False),
        scratch_types=[
            pltpu.VMEM((seqlen, CHUNK), jnp.int32),
            pltpu.VMEM((NBUF, CHUNK), jnp.int32),
            pltpu.VMEM((NBUF, CHUNK, D_MODEL), jnp.float32),
            pltpu.SemaphoreType.DMA,
            pltpu.SemaphoreType.DMA,
        ],
    )
    def gather_kernel(tokt_hbm, table_hbm, out_hbm, tok_v, idx_v, rows_v,
                      gsem, osem):
        wid = lax.axis_index("s") * NUM_CORES + lax.axis_index("c")
        pltpu.sync_copy(tokt_hbm.at[:, pl.ds(wid * bw, bw)], tok_v)
        ev2 = 2 * lax.iota(jnp.int32, 16)

        def group(g, _):
            q0 = g * (NBUF // 2)
            for b in range(NBUF):
                q = q0 + b // 2
                h = b % 2
                for r in range(2):
                    row = 2 * q + r
                    for gg in range(4):
                        src = tok_v[row, pl.ds(h * 64 + gg * 16, 16)]
                        plsc.store_scatter(
                            idx_v.at[b], [ev2 + (32 * gg + r)], src)
            gathers = []
            for b in range(NBUF):
                dma = pltpu.make_async_copy(
                    table_hbm.at[idx_v.at[b]], rows_v.at[b], gsem)
                dma.start()
                gathers.append(dma)
            outs = []
            for b in range(NBUF):
                j = g * NBUF + b
                q = q0 + b // 2
                h = b % 2
                gathers[b].wait()
                dma = pltpu.make_async_copy(
                    rows_v.at[b],
                    out_hbm.at[pl.ds(
                        (q * batch + wid * bw + h * 64) * 2, CHUNK)],
                    osem)
                dma.start()
                outs.append(dma)
            for b in range(NBUF):
                outs[b].wait()
            return 0

        lax.fori_loop(0, n_groups, group, 0)

    return gather_kernel


def _finisher_body(g_ref, s_ref, out_ref):
    for qi in range(Q_BLK):
        yt = g_ref[qi].T
        for si in range(2):
            scale = s_ref[2 * qi + si, :]
            out_ref[2 * qi + si, :, :] = (
                yt[si * D_MODEL:(si + 1) * D_MODEL, :] * scale[None, :])


def _finish(gathered, scale_sb, batch, seqlen, s_off, total_seqlen, donated):
    g4 = gathered.reshape(seqlen // 2, batch, 2 * D_MODEL)
    qb_off = s_off // (2 * Q_BLK)
    body = _finisher_body if donated is None else (
        lambda g_ref, s_ref, _, out_ref: _finisher_body(g_ref, s_ref, out_ref))
    in_specs = [
        pl.BlockSpec((Q_BLK, B_BLK, 2 * D_MODEL), lambda q, i: (q, i, 0)),
        pl.BlockSpec((2 * Q_BLK, B_BLK), lambda q, i: (q, i)),
    ]
    args = [g4, scale_sb]
    kwargs = {}
    if donated is not None:
        in_specs.append(pl.BlockSpec(memory_space=pl.ANY))
        args.append(donated)
        kwargs["input_output_aliases"] = {2: 0}
    return pl.pallas_call(
        body,
        grid=(seqlen // (2 * Q_BLK), batch // B_BLK),
        in_specs=in_specs,
        out_specs=pl.BlockSpec(
            (2 * Q_BLK, D_MODEL, B_BLK), lambda q, i: (q + qb_off, 0, i)),
        out_shape=jax.ShapeDtypeStruct((total_seqlen, D_MODEL, batch),
                                       jnp.float32),
        **kwargs,
    )(*args)


def kernel(tokens, table):
    batch, seqlen = tokens.shape
    tokens_t = tokens.T.astype(jnp.int32)
    scale_sb = jnp.where(tokens_t == 0, 0.0, jnp.float32(SCALE))
    splits = (0, 48, 96, 144, seqlen)
    out_t = None
    for s0, s1 in zip(splits[:-1], splits[1:]):
        sl = s1 - s0
        g = _make_gather(batch, sl)(tokens_t[s0:s1], table)
        out_t = _finish(g, scale_sb[s0:s1], batch, sl, s0, seqlen, out_t)
    return out_t.transpose(2, 0, 1)


# speedup vs baseline: 4.0339x; 1.1009x over previous
import functools
import math

import jax
import jax.numpy as jnp
from jax import lax
from jax.experimental import pallas as pl
from jax.experimental.pallas import tpu as pltpu
from jax.experimental.pallas import tpu_sc as plsc

D_MODEL = 64
SCALE = math.sqrt(D_MODEL)

NUM_CORES = 2
NUM_SUBCORES = 16
NUM_WORKERS = NUM_CORES * NUM_SUBCORES

CHUNK = 128
NBUF = 8

B_BLK = 1024
Q_BLK = 4


def _make_gather(batch, seqlen):
    num_tokens = batch * seqlen
    per_worker = num_tokens // NUM_WORKERS
    bw = batch // NUM_WORKERS
    assert bw == CHUNK and seqlen % 2 == 0
    n_chunks = per_worker // CHUNK
    assert NBUF % 2 == 0 and n_chunks % NBUF == 0
    n_groups = n_chunks // NBUF

    mesh = plsc.VectorSubcoreMesh(
        core_axis_name="c", subcore_axis_name="s",
        num_cores=NUM_CORES, num_subcores=NUM_SUBCORES)

    @functools.partial(
        pl.kernel,
        out_type=jax.ShapeDtypeStruct((num_tokens, D_MODEL), jnp.float32),
        mesh=mesh,
        compiler_params=pltpu.CompilerParams(
            use_tc_tiling_on_sc=False, needs_layout_passes=False),
        scratch_types=[
            pltpu.VMEM((seqlen, CHUNK), jnp.int32),
            pltpu.VMEM((NBUF, CHUNK), jnp.int32),
            pltpu.VMEM((NBUF, CHUNK, D_MODEL), jnp.float32),
            pltpu.SemaphoreType.DMA,
            pltpu.SemaphoreType.DMA,
        ],
    )
    def gather_kernel(tokt_hbm, table_hbm, out_hbm, tok_v, idx_v, rows_v,
                      gsem, osem):
        wid = lax.axis_index("s") * NUM_CORES + lax.axis_index("c")
        pltpu.sync_copy(tokt_hbm.at[:, pl.ds(wid * bw, bw)], tok_v)
        ev2 = 2 * lax.iota(jnp.int32, 16)

        def group(g, _):
            q0 = g * (NBUF // 2)
            for b in range(NBUF):
                q = q0 + b // 2
                h = b % 2
                for r in range(2):
                    row = 2 * q + r
                    for gg in range(4):
                        src = tok_v[row, pl.ds(h * 64 + gg * 16, 16)]
                        plsc.store_scatter(
                            idx_v.at[b], [ev2 + (32 * gg + r)], src)
            gathers = []
            for b in range(NBUF):
                dma = pltpu.make_async_copy(
                    table_hbm.at[idx_v.at[b]], rows_v.at[b], gsem)
                dma.start()
                gathers.append(dma)
            outs = []
            for b in range(NBUF):
                j = g * NBUF + b
                q = q0 + b // 2
                h = b % 2
                gathers[b].wait()
                dma = pltpu.make_async_copy(
                    rows_v.at[b],
                    out_hbm.at[pl.ds(
                        (q * batch + wid * bw + h * 64) * 2, CHUNK)],
                    osem)
                dma.start()
                outs.append(dma)
            for b in range(NBUF):
                outs[b].wait()
            return 0

        lax.fori_loop(0, n_groups, group, 0)

    return gather_kernel


def _finisher_body(g_ref, s_ref, out_ref):
    for qi in range(Q_BLK):
        yt = g_ref[qi].T
        for si in range(2):
            scale = s_ref[2 * qi + si, :]
            out_ref[2 * qi + si, :, :] = (
                yt[si * D_MODEL:(si + 1) * D_MODEL, :] * scale[None, :])


def _finish(gathered, scale_sb, batch, seqlen, s_off, total_seqlen, donated):
    g4 = gathered.reshape(seqlen // 2, batch, 2 * D_MODEL)
    qb_off = s_off // (2 * Q_BLK)
    body = _finisher_body if donated is None else (
        lambda g_ref, s_ref, _, out_ref: _finisher_body(g_ref, s_ref, out_ref))
    in_specs = [
        pl.BlockSpec((Q_BLK, B_BLK, 2 * D_MODEL), lambda q, i: (q, i, 0)),
        pl.BlockSpec((2 * Q_BLK, B_BLK), lambda q, i: (q, i)),
    ]
    args = [g4, scale_sb]
    kwargs = {}
    if donated is not None:
        in_specs.append(pl.BlockSpec(memory_space=pl.ANY))
        args.append(donated)
        kwargs["input_output_aliases"] = {2: 0}
    return pl.pallas_call(
        body,
        grid=(seqlen // (2 * Q_BLK), batch // B_BLK),
        in_specs=in_specs,
        out_specs=pl.BlockSpec(
            (2 * Q_BLK, D_MODEL, B_BLK), lambda q, i: (q + qb_off, 0, i)),
        out_shape=jax.ShapeDtypeStruct((total_seqlen, D_MODEL, batch),
                                       jnp.float32),
        **kwargs,
    )(*args)


def kernel(tokens, table):
    batch, seqlen = tokens.shape
    tokens_t = tokens.T.astype(jnp.int32)
    scale_sb = jnp.where(tokens_t == 0, 0.0, jnp.float32(SCALE))
    splits = (0, 48, 96, 144, seqlen)
    out_t = None
    for s0, s1 in zip(splits[:-1], splits[1:]):
        sl = s1 - s0
        g = _make_gather(batch, sl)(tokens_t[s0:s1], table)
        out_t = _finish(g, scale_sb[s0:s1], batch, sl, s0, seqlen, out_t)
    return out_t.transpose(2, 0, 1)


# speedup vs baseline: 4.1300x; 1.0238x over previous
import functools
import math

import jax
import jax.numpy as jnp
from jax import lax
from jax.experimental import pallas as pl
from jax.experimental.pallas import tpu as pltpu
from jax.experimental.pallas import tpu_sc as plsc

D_MODEL = 64
SCALE = math.sqrt(D_MODEL)

NUM_CORES = 2
NUM_SUBCORES = 16
NUM_WORKERS = NUM_CORES * NUM_SUBCORES

CHUNK = 128
NBUF = 8

B_BLK = 2048
Q_BLK = 4


def _make_gather(batch, seqlen):
    num_tokens = batch * seqlen
    per_worker = num_tokens // NUM_WORKERS
    bw = batch // NUM_WORKERS
    assert bw == CHUNK and seqlen % 2 == 0
    n_chunks = per_worker // CHUNK
    assert NBUF % 2 == 0 and n_chunks % NBUF == 0
    n_groups = n_chunks // NBUF

    mesh = plsc.VectorSubcoreMesh(
        core_axis_name="c", subcore_axis_name="s",
        num_cores=NUM_CORES, num_subcores=NUM_SUBCORES)

    @functools.partial(
        pl.kernel,
        out_type=jax.ShapeDtypeStruct((num_tokens, D_MODEL), jnp.float32),
        mesh=mesh,
        compiler_params=pltpu.CompilerParams(
            use_tc_tiling_on_sc=False, needs_layout_passes=False),
        scratch_types=[
            pltpu.VMEM((seqlen, CHUNK), jnp.int32),
            pltpu.VMEM((NBUF, CHUNK), jnp.int32),
            pltpu.VMEM((NBUF, CHUNK, D_MODEL), jnp.float32),
            pltpu.SemaphoreType.DMA,
            pltpu.SemaphoreType.DMA,
        ],
    )
    def gather_kernel(tokt_hbm, table_hbm, out_hbm, tok_v, idx_v, rows_v,
                      gsem, osem):
        wid = lax.axis_index("s") * NUM_CORES + lax.axis_index("c")
        pltpu.sync_copy(tokt_hbm.at[:, pl.ds(wid * bw, bw)], tok_v)
        ev2 = 2 * lax.iota(jnp.int32, 16)

        def group(g, _):
            q0 = g * (NBUF // 2)
            for b in range(NBUF):
                q = q0 + b // 2
                h = b % 2
                for r in range(2):
                    row = 2 * q + r
                    for gg in range(4):
                        src = tok_v[row, pl.ds(h * 64 + gg * 16, 16)]
                        plsc.store_scatter(
                            idx_v.at[b], [ev2 + (32 * gg + r)], src)
            gathers = []
            for b in range(NBUF):
                dma = pltpu.make_async_copy(
                    table_hbm.at[idx_v.at[b]], rows_v.at[b], gsem)
                dma.start()
                gathers.append(dma)
            outs = []
            for b in range(NBUF):
                j = g * NBUF + b
                q = q0 + b // 2
                h = b % 2
                gathers[b].wait()
                dma = pltpu.make_async_copy(
                    rows_v.at[b],
                    out_hbm.at[pl.ds(
                        (q * batch + wid * bw + h * 64) * 2, CHUNK)],
                    osem)
                dma.start()
                outs.append(dma)
            for b in range(NBUF):
                outs[b].wait()
            return 0

        lax.fori_loop(0, n_groups, group, 0)

    return gather_kernel


def _finisher_body(g_ref, s_ref, out_ref):
    for qi in range(Q_BLK):
        yt = g_ref[qi].T
        for si in range(2):
            scale = s_ref[2 * qi + si, :]
            out_ref[2 * qi + si, :, :] = (
                yt[si * D_MODEL:(si + 1) * D_MODEL, :] * scale[None, :])


def _finish(gathered, scale_sb, batch, seqlen, s_off, total_seqlen, donated):
    g4 = gathered.reshape(seqlen // 2, batch, 2 * D_MODEL)
    qb_off = s_off // (2 * Q_BLK)
    body = _finisher_body if donated is None else (
        lambda g_ref, s_ref, _, out_ref: _finisher_body(g_ref, s_ref, out_ref))
    in_specs = [
        pl.BlockSpec((Q_BLK, B_BLK, 2 * D_MODEL), lambda q, i: (q, i, 0)),
        pl.BlockSpec((2 * Q_BLK, B_BLK), lambda q, i: (q, i)),
    ]
    args = [g4, scale_sb]
    kwargs = {}
    if donated is not None:
        in_specs.append(pl.BlockSpec(memory_space=pl.ANY))
        args.append(donated)
        kwargs["input_output_aliases"] = {2: 0}
    return pl.pallas_call(
        body,
        grid=(seqlen // (2 * Q_BLK), batch // B_BLK),
        in_specs=in_specs,
        out_specs=pl.BlockSpec(
            (2 * Q_BLK, D_MODEL, B_BLK), lambda q, i: (q + qb_off, 0, i)),
        out_shape=jax.ShapeDtypeStruct((total_seqlen, D_MODEL, batch),
                                       jnp.float32),
        **kwargs,
    )(*args)


def kernel(tokens, table):
    batch, seqlen = tokens.shape
    tokens_t = tokens.T.astype(jnp.int32)
    scale_sb = jnp.where(tokens_t == 0, 0.0, jnp.float32(SCALE))
    splits = (0, 48, 96, 144, seqlen)
    out_t = None
    for s0, s1 in zip(splits[:-1], splits[1:]):
        sl = s1 - s0
        g = _make_gather(batch, sl)(tokens_t[s0:s1], table)
        out_t = _finish(g, scale_sb[s0:s1], batch, sl, s0, seqlen, out_t)
    return out_t.transpose(2, 0, 1)


# speedup vs baseline: 4.1741x; 1.0107x over previous
import functools
import math

import jax
import jax.numpy as jnp
from jax import lax
from jax.experimental import pallas as pl
from jax.experimental.pallas import tpu as pltpu
from jax.experimental.pallas import tpu_sc as plsc

D_MODEL = 64
SCALE = math.sqrt(D_MODEL)

NUM_CORES = 2
NUM_SUBCORES = 16
NUM_WORKERS = NUM_CORES * NUM_SUBCORES

CHUNK = 128
NBUF = 8

B_BLK = 4096
Q_BLK = 4


def _make_gather(batch, seqlen):
    num_tokens = batch * seqlen
    per_worker = num_tokens // NUM_WORKERS
    bw = batch // NUM_WORKERS
    assert bw == CHUNK and seqlen % 2 == 0
    n_chunks = per_worker // CHUNK
    assert NBUF % 2 == 0 and n_chunks % NBUF == 0
    n_groups = n_chunks // NBUF

    mesh = plsc.VectorSubcoreMesh(
        core_axis_name="c", subcore_axis_name="s",
        num_cores=NUM_CORES, num_subcores=NUM_SUBCORES)

    @functools.partial(
        pl.kernel,
        out_type=jax.ShapeDtypeStruct((num_tokens, D_MODEL), jnp.float32),
        mesh=mesh,
        compiler_params=pltpu.CompilerParams(
            use_tc_tiling_on_sc=False, needs_layout_passes=False),
        scratch_types=[
            pltpu.VMEM((seqlen, CHUNK), jnp.int32),
            pltpu.VMEM((NBUF, CHUNK), jnp.int32),
            pltpu.VMEM((NBUF, CHUNK, D_MODEL), jnp.float32),
            pltpu.SemaphoreType.DMA,
            pltpu.SemaphoreType.DMA,
        ],
    )
    def gather_kernel(tokt_hbm, table_hbm, out_hbm, tok_v, idx_v, rows_v,
                      gsem, osem):
        wid = lax.axis_index("s") * NUM_CORES + lax.axis_index("c")
        pltpu.sync_copy(tokt_hbm.at[:, pl.ds(wid * bw, bw)], tok_v)
        ev2 = 2 * lax.iota(jnp.int32, 16)

        def group(g, _):
            q0 = g * (NBUF // 2)
            for b in range(NBUF):
                q = q0 + b // 2
                h = b % 2
                for r in range(2):
                    row = 2 * q + r
                    for gg in range(4):
                        src = tok_v[row, pl.ds(h * 64 + gg * 16, 16)]
                        plsc.store_scatter(
                            idx_v.at[b], [ev2 + (32 * gg + r)], src)
            gathers = []
            for b in range(NBUF):
                dma = pltpu.make_async_copy(
                    table_hbm.at[idx_v.at[b]], rows_v.at[b], gsem)
                dma.start()
                gathers.append(dma)
            outs = []
            for b in range(NBUF):
                j = g * NBUF + b
                q = q0 + b // 2
                h = b % 2
                gathers[b].wait()
                dma = pltpu.make_async_copy(
                    rows_v.at[b],
                    out_hbm.at[pl.ds(
                        (q * batch + wid * bw + h * 64) * 2, CHUNK)],
                    osem)
                dma.start()
                outs.append(dma)
            for b in range(NBUF):
                outs[b].wait()
            return 0

        lax.fori_loop(0, n_groups, group, 0)

    return gather_kernel


def _finisher_body(g_ref, s_ref, out_ref):
    for qi in range(Q_BLK):
        yt = g_ref[qi].T
        for si in range(2):
            scale = s_ref[2 * qi + si, :]
            out_ref[2 * qi + si, :, :] = (
                yt[si * D_MODEL:(si + 1) * D_MODEL, :] * scale[None, :])


def _finish(gathered, scale_sb, batch, seqlen, s_off, total_seqlen, donated):
    g4 = gathered.reshape(seqlen // 2, batch, 2 * D_MODEL)
    qb_off = s_off // (2 * Q_BLK)
    body = _finisher_body if donated is None else (
        lambda g_ref, s_ref, _, out_ref: _finisher_body(g_ref, s_ref, out_ref))
    in_specs = [
        pl.BlockSpec((Q_BLK, B_BLK, 2 * D_MODEL), lambda q, i: (q, i, 0)),
        pl.BlockSpec((2 * Q_BLK, B_BLK), lambda q, i: (q, i)),
    ]
    args = [g4, scale_sb]
    kwargs = {}
    if donated is not None:
        in_specs.append(pl.BlockSpec(memory_space=pl.ANY))
        args.append(donated)
        kwargs["input_output_aliases"] = {2: 0}
    return pl.pallas_call(
        body,
        grid=(seqlen // (2 * Q_BLK), batch // B_BLK),
        in_specs=in_specs,
        out_specs=pl.BlockSpec(
            (2 * Q_BLK, D_MODEL, B_BLK), lambda q, i: (q + qb_off, 0, i)),
        out_shape=jax.ShapeDtypeStruct((total_seqlen, D_MODEL, batch),
                                       jnp.float32),
        **kwargs,
    )(*args)


def kernel(tokens, table):
    batch, seqlen = tokens.shape
    tokens_t = tokens.T.astype(jnp.int32)
    scale_sb = jnp.where(tokens_t == 0, 0.0, jnp.float32(SCALE))
    splits = (0, 48, 96, 144, seqlen)
    out_t = None
    for s0, s1 in zip(splits[:-1], splits[1:]):
        sl = s1 - s0
        g = _make_gather(batch, sl)(tokens_t[s0:s1], table)
        out_t = _finish(g, scale_sb[s0:s1], batch, sl, s0, seqlen, out_t)
    return out_t.transpose(2, 0, 1)
